# Initial kernel scaffold; baseline (speedup 1.0000x reference)
import jax
import jax.numpy as jnp
from jax.experimental import pallas as pl


def kernel(iword, owords, nwords, emb_in, emb_out):
    raise NotImplementedError("write your pallas kernel here")


# speedup vs baseline: 1.1955x; 1.1955x over previous
import functools
import jax
import jax.numpy as jnp
from jax import lax
from jax.experimental import pallas as pl
from jax.experimental.pallas import tpu as pltpu
from jax.experimental.pallas import tpu_sc as plsc

NW = 32
LANES = 16
CHUNK_B = 8


def _sc_scores(words, iv, emb_out, slots):
    B = iv.shape[0]
    dim = iv.shape[1]
    per_w = B // NW
    n_chunks = per_w // CHUNK_B
    cw = CHUNK_B * slots
    groups = slots // LANES

    mesh = plsc.VectorSubcoreMesh(core_axis_name="c", subcore_axis_name="s")

    @functools.partial(
        pl.kernel,
        mesh=mesh,
        compiler_params=pltpu.CompilerParams(
            needs_layout_passes=False, use_tc_tiling_on_sc=False),
        out_type=jax.ShapeDtypeStruct((B * slots,), jnp.float32),
        scratch_types=[
            pltpu.VMEM((cw,), jnp.int32),
            pltpu.VMEM((cw, dim), jnp.float32),
            pltpu.VMEM((CHUNK_B, dim), jnp.float32),
            pltpu.VMEM((cw,), jnp.float32),
            pltpu.SemaphoreType.DMA,
        ],
    )
    def k(words_hbm, iv_hbm, eout_hbm, out_hbm,
          widx_v, rows_v, iv_v, scores_v, sem_r):
        wid = lax.axis_index("s") * 2 + lax.axis_index("c")

        def chunk_body(c, carry):
            base_b = wid * per_w + c * CHUNK_B
            pltpu.sync_copy(words_hbm.at[pl.ds(base_b * slots, cw)], widx_v)
            pltpu.sync_copy(iv_hbm.at[pl.ds(base_b, CHUNK_B), :], iv_v)
            cp_r = pltpu.async_copy(eout_hbm.at[widx_v], rows_v, sem_r)
            cp_r.wait()

            def b_body(bl, carry2):
                blv = jnp.full((LANES,), bl, jnp.int32)
                iv_splats = [
                    plsc.load_gather(iv_v, [blv, jnp.full((LANES,), d,
                                                          jnp.int32)])
                    for d in range(dim)
                ]

                def g_body(g, carry3):
                    slot = bl * slots + g * LANES
                    rowv = slot + lax.iota(jnp.int32, LANES)
                    acc = jnp.zeros((LANES,), jnp.float32)
                    for d in range(dim):
                        col = jnp.full((LANES,), d, jnp.int32)
                        vals = plsc.load_gather(rows_v, [rowv, col])
                        acc = acc + vals * iv_splats[d]
                    scores_v[pl.ds(slot, LANES)] = acc
                    return carry3

                return lax.fori_loop(0, groups, g_body, carry2)

            lax.fori_loop(0, CHUNK_B, b_body, carry)
            pltpu.sync_copy(scores_v, out_hbm.at[pl.ds(base_b * slots, cw)])
            return carry

        lax.fori_loop(0, n_chunks, chunk_body, 0)

    return k(words, iv, emb_out)


def _tc_loss(scores2d, signb, denom):
    def body(s_ref, sb_ref, o_ref):
        x = s_ref[...]
        z = x * sb_ref[0, :][None, :] + sb_ref[1, :][None, :]
        sp = jnp.maximum(z, 0.0) + jnp.log1p(jnp.exp(-jnp.abs(z)))
        o_ref[0, 0] = jnp.sum(sp) * (1.0 / denom)

    out = pl.pallas_call(
        body,
        out_shape=jax.ShapeDtypeStruct((1, 1), jnp.float32),
        out_specs=pl.BlockSpec(memory_space=pltpu.SMEM),
    )(scores2d, signb)
    return out[0, 0]


def kernel(iword, owords, nwords, emb_in, emb_out):
    B = iword.shape[0]
    ctx = owords.shape[1]
    nneg = nwords.shape[1]
    nvalid = ctx + nneg
    slots = -(-nvalid // LANES) * LANES

    words = jnp.concatenate(
        [owords.astype(jnp.int32), nwords.astype(jnp.int32),
         jnp.zeros((B, slots - nvalid), jnp.int32)], axis=1)
    iv = jnp.take(emb_in, iword, axis=0)
    scores = _sc_scores(words.reshape(-1), iv, emb_out, slots)

    sign = jnp.concatenate([
        jnp.full((ctx,), -1.0, jnp.float32),
        jnp.ones((nneg,), jnp.float32),
        jnp.zeros((slots - nvalid,), jnp.float32)])
    bias = jnp.concatenate([
        jnp.zeros((nvalid,), jnp.float32),
        jnp.full((slots - nvalid,), -1e9, jnp.float32)])
    signb = jnp.stack([sign, bias])

    return _tc_loss(scores.reshape(B, slots), signb, float(B * ctx))


# speedup vs baseline: 1.2223x; 1.0224x over previous
import functools
import jax
import jax.numpy as jnp
from jax import lax
from jax.experimental import pallas as pl
from jax.experimental.pallas import tpu as pltpu
from jax.experimental.pallas import tpu_sc as plsc

NW = 32
LANES = 16
CHUNK_B = 8


def _sc_scores(words, iv, emb_out, slots):
    B = iv.shape[0]
    dim = iv.shape[1]
    per_w = B // NW
    n_chunks = per_w // CHUNK_B
    cw = CHUNK_B * slots
    groups = slots // LANES

    mesh = plsc.VectorSubcoreMesh(core_axis_name="c", subcore_axis_name="s")

    @functools.partial(
        pl.kernel,
        mesh=mesh,
        compiler_params=pltpu.CompilerParams(
            needs_layout_passes=False, use_tc_tiling_on_sc=False),
        out_type=jax.ShapeDtypeStruct((B * slots,), jnp.float32),
        scratch_types=[
            pltpu.VMEM((2, cw), jnp.int32),
            pltpu.VMEM((2, cw, dim), jnp.float32),
            pltpu.VMEM((2, CHUNK_B, dim), jnp.float32),
            pltpu.VMEM((cw,), jnp.float32),
            pltpu.SemaphoreType.DMA,
            pltpu.SemaphoreType.DMA,
        ],
    )
    def k(words_hbm, iv_hbm, eout_hbm, out_hbm,
          widx_v, rows_v, iv_v, scores_v, sem0, sem1):
        wid = lax.axis_index("s") * 2 + lax.axis_index("c")
        sems = (sem0, sem1)

        def issue(c):
            p = c % 2
            base_b = wid * per_w + c * CHUNK_B
            pltpu.sync_copy(words_hbm.at[pl.ds(base_b * slots, cw)],
                            widx_v.at[p])
            pltpu.sync_copy(iv_hbm.at[pl.ds(base_b, CHUNK_B), :], iv_v.at[p])
            return pltpu.async_copy(eout_hbm.at[widx_v.at[p]], rows_v.at[p],
                                    sems[p])

        def compute(c):
            p = c % 2
            base_b = wid * per_w + c * CHUNK_B
            rows_p = rows_v.at[p]

            def b_body(bl, carry2):
                blv = jnp.full((LANES,), bl, jnp.int32)
                iv_splats = [
                    plsc.load_gather(iv_v.at[p],
                                     [blv, jnp.full((LANES,), d, jnp.int32)])
                    for d in range(dim)
                ]

                def g_body(g, carry3):
                    slot = bl * slots + g * LANES
                    rowv = slot + lax.iota(jnp.int32, LANES)
                    acc = jnp.zeros((LANES,), jnp.float32)
                    for d in range(dim):
                        col = jnp.full((LANES,), d, jnp.int32)
                        vals = plsc.load_gather(rows_p, [rowv, col])
                        acc = acc + vals * iv_splats[d]
                    scores_v[pl.ds(slot, LANES)] = acc
                    return carry3

                return lax.fori_loop(0, groups, g_body, carry2)

            lax.fori_loop(0, CHUNK_B, b_body, 0)
            pltpu.sync_copy(scores_v, out_hbm.at[pl.ds(base_b * slots, cw)])

        descs = {0: issue(0)}
        for c in range(n_chunks):
            if c + 1 < n_chunks:
                descs[c + 1] = issue(c + 1)
            descs[c].wait()
            compute(c)

    return k(words, iv, emb_out)


def _tc_loss(scores2d, signb, denom):
    def body(s_ref, sb_ref, o_ref):
        x = s_ref[...]
        z = x * sb_ref[0, :][None, :] + sb_ref[1, :][None, :]
        sp = jnp.maximum(z, 0.0) + jnp.log1p(jnp.exp(-jnp.abs(z)))
        o_ref[0, 0] = jnp.sum(sp) * (1.0 / denom)

    out = pl.pallas_call(
        body,
        out_shape=jax.ShapeDtypeStruct((1, 1), jnp.float32),
        out_specs=pl.BlockSpec(memory_space=pltpu.SMEM),
    )(scores2d, signb)
    return out[0, 0]


def kernel(iword, owords, nwords, emb_in, emb_out):
    B = iword.shape[0]
    ctx = owords.shape[1]
    nneg = nwords.shape[1]
    nvalid = ctx + nneg
    slots = -(-nvalid // LANES) * LANES

    words = jnp.concatenate(
        [owords.astype(jnp.int32), nwords.astype(jnp.int32),
         jnp.zeros((B, slots - nvalid), jnp.int32)], axis=1)
    iv = jnp.take(emb_in, iword, axis=0)
    scores = _sc_scores(words.reshape(-1), iv, emb_out, slots)

    sign = jnp.concatenate([
        jnp.full((ctx,), -1.0, jnp.float32),
        jnp.ones((nneg,), jnp.float32),
        jnp.zeros((slots - nvalid,), jnp.float32)])
    bias = jnp.concatenate([
        jnp.zeros((nvalid,), jnp.float32),
        jnp.full((slots - nvalid,), -1e9, jnp.float32)])
    signb = jnp.stack([sign, bias])

    return _tc_loss(scores.reshape(B, slots), signb, float(B * ctx))


# speedup vs baseline: 1.2237x; 1.0011x over previous
import functools
import jax
import jax.numpy as jnp
from jax import lax
from jax.experimental import pallas as pl
from jax.experimental.pallas import tpu as pltpu
from jax.experimental.pallas import tpu_sc as plsc

NW = 32
LANES = 16
CHUNK_B = 8


def _sc_scores(words, iv, emb_out, slots):
    B = iv.shape[0]
    dim = iv.shape[1]
    per_w = B // NW
    n_chunks = per_w // CHUNK_B
    cw = CHUNK_B * slots
    groups = slots // LANES

    mesh = plsc.VectorSubcoreMesh(core_axis_name="c", subcore_axis_name="s")

    @functools.partial(
        pl.kernel,
        mesh=mesh,
        compiler_params=pltpu.CompilerParams(
            needs_layout_passes=False, use_tc_tiling_on_sc=False),
        out_type=jax.ShapeDtypeStruct((B * slots,), jnp.float32),
        scratch_types=[
            pltpu.VMEM((2, cw), jnp.int32),
            pltpu.VMEM((2, cw, dim), jnp.float32),
            pltpu.VMEM((2, CHUNK_B, dim), jnp.float32),
            pltpu.VMEM((2, cw), jnp.float32),
            pltpu.SemaphoreType.DMA,
            pltpu.SemaphoreType.DMA,
            pltpu.SemaphoreType.DMA,
        ],
    )
    def k(words_hbm, iv_hbm, eout_hbm, out_hbm,
          widx_v, rows_v, iv_v, scores_v, sem0, sem1, sem_w):
        wid = lax.axis_index("s") * 2 + lax.axis_index("c")
        sems = (sem0, sem1)

        def issue(c):
            p = c % 2
            base_b = wid * per_w + c * CHUNK_B
            pltpu.sync_copy(words_hbm.at[pl.ds(base_b * slots, cw)],
                            widx_v.at[p])
            pltpu.sync_copy(iv_hbm.at[pl.ds(base_b, CHUNK_B), :], iv_v.at[p])
            return pltpu.async_copy(eout_hbm.at[widx_v.at[p]], rows_v.at[p],
                                    sems[p])

        def compute(c):
            p = c % 2
            base_b = wid * per_w + c * CHUNK_B
            rows_p = rows_v.at[p]

            def b_body(bl, carry2):
                blv = jnp.full((LANES,), bl, jnp.int32)
                iv_splats = [
                    plsc.load_gather(iv_v.at[p],
                                     [blv, jnp.full((LANES,), d, jnp.int32)])
                    for d in range(dim)
                ]

                def g_body(g, carry3):
                    slot = bl * slots + g * LANES
                    rowv = slot + lax.iota(jnp.int32, LANES)
                    acc = jnp.zeros((LANES,), jnp.float32)
                    for d in range(dim):
                        col = jnp.full((LANES,), d, jnp.int32)
                        vals = plsc.load_gather(rows_p, [rowv, col])
                        acc = acc + vals * iv_splats[d]
                    scores_v[p, pl.ds(slot, LANES)] = acc
                    return carry3

                return lax.fori_loop(0, groups, g_body, carry2)

            lax.fori_loop(0, CHUNK_B, b_body, 0)
            return pltpu.async_copy(
                scores_v.at[p], out_hbm.at[pl.ds(base_b * slots, cw)], sem_w)

        descs = {0: issue(0)}
        wb = {}
        for c in range(n_chunks):
            if c + 1 < n_chunks:
                descs[c + 1] = issue(c + 1)
            descs[c].wait()
            if c >= 2:
                wb[c - 2].wait()
            wb[c] = compute(c)
        wb[n_chunks - 2].wait()
        wb[n_chunks - 1].wait()

    return k(words, iv, emb_out)


def _tc_loss(scores2d, signb, denom):
    def body(s_ref, sb_ref, o_ref):
        x = s_ref[...]
        z = x * sb_ref[0, :][None, :] + sb_ref[1, :][None, :]
        sp = jnp.maximum(z, 0.0) + jnp.log1p(jnp.exp(-jnp.abs(z)))
        o_ref[0, 0] = jnp.sum(sp) * (1.0 / denom)

    out = pl.pallas_call(
        body,
        out_shape=jax.ShapeDtypeStruct((1, 1), jnp.float32),
        out_specs=pl.BlockSpec(memory_space=pltpu.SMEM),
    )(scores2d, signb)
    return out[0, 0]


def kernel(iword, owords, nwords, emb_in, emb_out):
    B = iword.shape[0]
    ctx = owords.shape[1]
    nneg = nwords.shape[1]
    nvalid = ctx + nneg
    slots = -(-nvalid // LANES) * LANES

    words = jnp.concatenate(
        [owords.astype(jnp.int32), nwords.astype(jnp.int32),
         jnp.zeros((B, slots - nvalid), jnp.int32)], axis=1)
    iv = jnp.take(emb_in, iword, axis=0)
    scores = _sc_scores(words.reshape(-1), iv, emb_out, slots)

    sign = jnp.concatenate([
        jnp.full((ctx,), -1.0, jnp.float32),
        jnp.ones((nneg,), jnp.float32),
        jnp.zeros((slots - nvalid,), jnp.float32)])
    bias = jnp.concatenate([
        jnp.zeros((nvalid,), jnp.float32),
        jnp.full((slots - nvalid,), -1e9, jnp.float32)])
    signb = jnp.stack([sign, bias])

    return _tc_loss(scores.reshape(B, slots), signb, float(B * ctx))


# speedup vs baseline: 1.4489x; 1.1840x over previous
import functools
import jax
import jax.numpy as jnp
from jax import lax
from jax.experimental import pallas as pl
from jax.experimental.pallas import tpu as pltpu
from jax.experimental.pallas import tpu_sc as plsc

NW = 32
LANES = 16
CHUNK_B = 8


def _sc_scores(words, iv, emb_out, slots):
    B = iv.shape[0]
    dim = iv.shape[1]
    per_w = B // NW
    n_chunks = per_w // CHUNK_B
    cw = CHUNK_B * slots
    groups = cw // LANES

    mesh = plsc.VectorSubcoreMesh(core_axis_name="c", subcore_axis_name="s")

    @functools.partial(
        pl.kernel,
        mesh=mesh,
        compiler_params=pltpu.CompilerParams(
            needs_layout_passes=False, use_tc_tiling_on_sc=False),
        out_type=jax.ShapeDtypeStruct((B * slots,), jnp.float32),
        scratch_types=[
            pltpu.VMEM((2, cw), jnp.int32),
            pltpu.VMEM((2, cw, dim), jnp.float32),
            pltpu.VMEM((2, CHUNK_B, dim), jnp.float32),
            pltpu.VMEM((2, cw), jnp.float32),
            pltpu.SemaphoreType.DMA,
            pltpu.SemaphoreType.DMA,
            pltpu.SemaphoreType.DMA,
        ],
    )
    def k(words_hbm, iv_hbm, eout_hbm, out_hbm,
          widx_v, rows_v, iv_v, scores_v, sem0, sem1, sem_w):
        wid = lax.axis_index("s") * 2 + lax.axis_index("c")
        sems = (sem0, sem1)

        def issue(c):
            p = c % 2
            base_b = wid * per_w + c * CHUNK_B
            pltpu.sync_copy(words_hbm.at[pl.ds(base_b * slots, cw)],
                            widx_v.at[p])
            pltpu.sync_copy(iv_hbm.at[pl.ds(base_b, CHUNK_B), :], iv_v.at[p])
            return pltpu.async_copy(eout_hbm.at[widx_v.at[p]], rows_v.at[p],
                                    sems[p])

        def compute(c):
            p = c % 2
            base_b = wid * per_w + c * CHUNK_B
            rows_p = rows_v.at[p]
            iv_p = iv_v.at[p]

            def g_body(g, carry2):
                slot = g * LANES
                rowv = slot + lax.iota(jnp.int32, LANES)
                bvec = rowv // slots
                acc = jnp.zeros((LANES,), jnp.float32)
                for d in range(dim):
                    col = jnp.full((LANES,), d, jnp.int32)
                    vals = plsc.load_gather(rows_p, [rowv, col])
                    ivd = plsc.load_gather(iv_p, [bvec, col])
                    acc = acc + vals * ivd
                scores_v[p, pl.ds(slot, LANES)] = acc
                return carry2

            lax.fori_loop(0, groups, g_body, 0)
            return pltpu.async_copy(
                scores_v.at[p], out_hbm.at[pl.ds(base_b * slots, cw)], sem_w)

        descs = {0: issue(0)}
        wb = {}
        for c in range(n_chunks):
            if c + 1 < n_chunks:
                descs[c + 1] = issue(c + 1)
            descs[c].wait()
            if c >= 2:
                wb[c - 2].wait()
            wb[c] = compute(c)
        wb[n_chunks - 2].wait()
        wb[n_chunks - 1].wait()

    return k(words, iv, emb_out)


def _tc_loss(scores2d, sign, denom):
    def body(s_ref, sg_ref, o_ref):
        z = s_ref[...] * sg_ref[0, :][None, :]
        sp = jnp.maximum(z, 0.0) + jnp.log1p(jnp.exp(-jnp.abs(z)))
        o_ref[0, 0] = jnp.sum(sp) * (1.0 / denom)

    out = pl.pallas_call(
        body,
        out_shape=jax.ShapeDtypeStruct((1, 1), jnp.float32),
        out_specs=pl.BlockSpec(memory_space=pltpu.SMEM),
    )(scores2d, sign)
    return out[0, 0]


def kernel(iword, owords, nwords, emb_in, emb_out):
    B = iword.shape[0]
    ctx = owords.shape[1]
    nneg = nwords.shape[1]
    slots = ctx + nneg

    words = jnp.concatenate(
        [owords.astype(jnp.int32), nwords.astype(jnp.int32)], axis=1)
    iv = jnp.take(emb_in, iword, axis=0)
    scores = _sc_scores(words.reshape(-1), iv, emb_out, slots)

    sign = jnp.concatenate([
        jnp.full((ctx,), -1.0, jnp.float32),
        jnp.ones((nneg,), jnp.float32)])[None, :]

    return _tc_loss(scores.reshape(B, slots), sign, float(B * ctx))
